# baseline (speedup 1.0000x reference)
import functools

import jax
import jax.numpy as jnp
from jax import lax
from jax.experimental import pallas as pl
from jax.experimental.pallas import tpu as pltpu
from jax.experimental.pallas import tpu_sc as plsc

_NC = 2
_NS = 16
_NW = _NC * _NS
_R = 16
_L = 16
_U = 8


def _make_sc_add(B, S, D, dtype):
    rows_per_w = S // _NW
    cw = _R * D
    n_iters = cw // (_L * _U)

    @functools.partial(
        pl.kernel,
        mesh=plsc.VectorSubcoreMesh(core_axis_name="c", subcore_axis_name="s"),
        out_type=jax.ShapeDtypeStruct((B * S * D,), dtype),
        scratch_types=[
            pltpu.VMEM((cw,), dtype),
            pltpu.VMEM((cw,), dtype),
        ],
    )
    def sc_add(x_hbm, pos_hbm, out_hbm, pbuf, buf):
        wid = lax.axis_index("s") * _NC + lax.axis_index("c")
        base = wid * rows_per_w

        def chunk(i, carry):
            off = (base + i * _R) * D
            pltpu.sync_copy(pos_hbm.at[pl.ds(off, cw)], pbuf)
            for b in range(B):
                xoff = b * S * D + off
                pltpu.sync_copy(x_hbm.at[pl.ds(xoff, cw)], buf)

                def add_body(k, c):
                    kb = k * (_L * _U)
                    for u in range(_U):
                        sl = pl.ds(kb + u * _L, _L)
                        plsc.addupdate(buf.at[sl], pbuf[sl])
                    return c

                lax.fori_loop(0, n_iters, add_body, 0)
                pltpu.sync_copy(buf, out_hbm.at[pl.ds(xoff, cw)])
            return carry

        lax.fori_loop(0, rows_per_w // _R, chunk, 0)

    return sc_add


def kernel(x, pos_table):
    B, S, D = x.shape
    out = _make_sc_add(B, S, D, x.dtype)(
        x.reshape(B * S * D), pos_table.reshape(S * D)
    )
    return out.reshape(B, S, D)
